# speedup vs baseline: 2.6667x; 2.6667x over previous
import functools

import jax
import jax.numpy as jnp
from jax import lax
from jax.experimental import pallas as pl
from jax.experimental.pallas import tpu as pltpu
from jax.experimental.pallas import tpu_sc as plsc

LAT_BINS = 1800
LON_BINS = 3600
N_EMBD = 384
B = 4096
L = 50

NC, NS, LANES = 2, 16, 16
NW = NC * NS
N = B * L
PER_W = N // NW
CHUNK = 128
NCHUNK = PER_W // CHUNK

_mesh = plsc.VectorSubcoreMesh(core_axis_name="c", subcore_axis_name="s")


@functools.partial(
    pl.kernel,
    out_type=jax.ShapeDtypeStruct((N, N_EMBD), jnp.float32),
    mesh=_mesh,
    scratch_types=[
        pltpu.VMEM((CHUNK,), jnp.float32),
        pltpu.VMEM((CHUNK,), jnp.int32),
        pltpu.VMEM((CHUNK,), jnp.int32),
        pltpu.VMEM((CHUNK, N_EMBD), jnp.float32),
        pltpu.VMEM((CHUNK, N_EMBD), jnp.float32),
        pltpu.SemaphoreType.DMA,
        pltpu.SemaphoreType.DMA,
    ],
)
def _sc_embed(lat_hbm, lon_hbm, lat_tab, lon_tab, out_hbm,
              coords_v, ilat_v, ilon_v, rows_a, rows_b, sem_a, sem_b):
    wid = lax.axis_index("s") * NC + lax.axis_index("c")
    base = wid * PER_W

    def chunk_body(ci, carry):
        off = base + ci * CHUNK

        pltpu.sync_copy(lat_hbm.at[pl.ds(off, CHUNK)], coords_v)
        for j in range(CHUNK // LANES):
            c = coords_v[pl.ds(j * LANES, LANES)]
            i = ((c + 90.0) * (LAT_BINS / 180.0)).astype(jnp.int32)
            ilat_v[pl.ds(j * LANES, LANES)] = jnp.minimum(
                jnp.maximum(i, 0), LAT_BINS - 1)

        pltpu.sync_copy(lon_hbm.at[pl.ds(off, CHUNK)], coords_v)
        for j in range(CHUNK // LANES):
            c = coords_v[pl.ds(j * LANES, LANES)]
            i = ((c + 180.0) * (LON_BINS / 360.0)).astype(jnp.int32)
            ilon_v[pl.ds(j * LANES, LANES)] = jnp.minimum(
                jnp.maximum(i, 0), LON_BINS - 1)

        cp_a = pltpu.async_copy(lat_tab.at[ilat_v], rows_a, sem_a)
        cp_b = pltpu.async_copy(lon_tab.at[ilon_v], rows_b, sem_b)
        cp_a.wait()
        cp_b.wait()

        def add_row(r, carry2):
            for j in range(N_EMBD // LANES):
                sl = pl.ds(j * LANES, LANES)
                rows_a[r, sl] = rows_a[r, sl] + rows_b[r, sl]
            return carry2

        lax.fori_loop(0, CHUNK, add_row, 0, unroll=False)

        pltpu.sync_copy(rows_a, out_hbm.at[pl.ds(off, CHUNK)])
        return carry

    lax.fori_loop(0, NCHUNK, chunk_body, 0, unroll=False)


def kernel(lat, lon, lat_table, lon_table):
    lat_flat = lat.reshape(N)
    lon_flat = lon.reshape(N)
    out = _sc_embed(lat_flat, lon_flat, lat_table, lon_table)
    return out.reshape(B, L, N_EMBD)


# speedup vs baseline: 3.3772x; 1.2665x over previous
import functools

import jax
import jax.numpy as jnp
from jax import lax
from jax.experimental import pallas as pl
from jax.experimental.pallas import tpu as pltpu
from jax.experimental.pallas import tpu_sc as plsc

LAT_BINS = 1800
LON_BINS = 3600
N_EMBD = 384
B = 4096
L = 50

NC, NS, LANES = 2, 16, 16
NW = NC * NS
N = B * L
PER_W = N // NW
CHUNK = 64
NCHUNK = PER_W // CHUNK

_mesh = plsc.VectorSubcoreMesh(core_axis_name="c", subcore_axis_name="s")


@functools.partial(
    pl.kernel,
    out_type=jax.ShapeDtypeStruct((N, N_EMBD), jnp.float32),
    mesh=_mesh,
    scratch_types=[
        pltpu.VMEM((PER_W,), jnp.float32),
        pltpu.VMEM((PER_W,), jnp.int32),
        pltpu.VMEM((PER_W,), jnp.int32),
        pltpu.VMEM((CHUNK, N_EMBD), jnp.float32),
        pltpu.VMEM((CHUNK, N_EMBD), jnp.float32),
        pltpu.VMEM((CHUNK, N_EMBD), jnp.float32),
        pltpu.VMEM((CHUNK, N_EMBD), jnp.float32),
        pltpu.SemaphoreType.DMA,
        pltpu.SemaphoreType.DMA,
        pltpu.SemaphoreType.DMA,
        pltpu.SemaphoreType.DMA,
        pltpu.SemaphoreType.DMA,
        pltpu.SemaphoreType.DMA,
    ],
)
def _sc_embed(lat_hbm, lon_hbm, lat_tab, lon_tab, out_hbm,
              coords_v, ilat_v, ilon_v,
              ga0, ga1, gb0, gb1,
              sga0, sga1, sgb0, sgb1, swb0, swb1):
    wid = lax.axis_index("s") * NC + lax.axis_index("c")
    base = wid * PER_W

    ga = (ga0, ga1)
    gb = (gb0, gb1)
    sga = (sga0, sga1)
    sgb = (sgb0, sgb1)
    swb = (swb0, swb1)

    pltpu.sync_copy(lat_hbm.at[pl.ds(base, PER_W)], coords_v)

    def lat_idx_body(g, carry):
        sl = pl.ds(g * LANES, LANES)
        i = ((coords_v[sl] + 90.0) * (LAT_BINS / 180.0)).astype(jnp.int32)
        ilat_v[sl] = jnp.minimum(jnp.maximum(i, 0), LAT_BINS - 1)
        return carry

    lax.fori_loop(0, PER_W // LANES, lat_idx_body, 0, unroll=False)

    pltpu.sync_copy(lon_hbm.at[pl.ds(base, PER_W)], coords_v)

    def lon_idx_body(g, carry):
        sl = pl.ds(g * LANES, LANES)
        i = ((coords_v[sl] + 180.0) * (LON_BINS / 360.0)).astype(jnp.int32)
        ilon_v[sl] = jnp.minimum(jnp.maximum(i, 0), LON_BINS - 1)
        return carry

    lax.fori_loop(0, PER_W // LANES, lon_idx_body, 0, unroll=False)

    def issue_gathers(ci, b):
        isl = pl.ds(ci * CHUNK, CHUNK)
        pltpu.async_copy(lat_tab.at[ilat_v.at[isl]], ga[b], sga[b])
        pltpu.async_copy(lon_tab.at[ilon_v.at[isl]], gb[b], sgb[b])

    def wait_gathers(b):
        pltpu.make_async_copy(lat_tab.at[pl.ds(0, CHUNK)], ga[b], sga[b]).wait()
        pltpu.make_async_copy(lon_tab.at[pl.ds(0, CHUNK)], gb[b], sgb[b]).wait()

    def issue_wb(ci, b):
        pltpu.async_copy(ga[b], out_hbm.at[pl.ds(base + ci * CHUNK, CHUNK)],
                         swb[b])

    def wait_wb(b):
        pltpu.make_async_copy(ga[b], out_hbm.at[pl.ds(0, CHUNK)], swb[b]).wait()

    issue_gathers(0, 0)

    def pair_body(pi, carry):
        for b in (0, 1):
            ci = pi * 2 + b
            nxt = ci + 1
            other = 1 - b

            @pl.when(nxt < NCHUNK)
            def _issue_next():
                @pl.when(nxt >= 2)
                def _drain_wb():
                    wait_wb(other)

                issue_gathers(nxt, other)

            wait_gathers(b)

            def add_row(r, carry2):
                for j in range(N_EMBD // LANES):
                    sl = pl.ds(j * LANES, LANES)
                    ga[b][r, sl] = ga[b][r, sl] + gb[b][r, sl]
                return carry2

            lax.fori_loop(0, CHUNK, add_row, 0, unroll=False)

            issue_wb(ci, b)
        return carry

    lax.fori_loop(0, NCHUNK // 2, pair_body, 0, unroll=False)

    wait_wb(0)
    wait_wb(1)


def kernel(lat, lon, lat_table, lon_table):
    lat_flat = lat.reshape(N)
    lon_flat = lon.reshape(N)
    out = _sc_embed(lat_flat, lon_flat, lat_table, lon_table)
    return out.reshape(B, L, N_EMBD)


# speedup vs baseline: 3.3825x; 1.0016x over previous
import functools

import jax
import jax.numpy as jnp
from jax import lax
from jax.experimental import pallas as pl
from jax.experimental.pallas import tpu as pltpu
from jax.experimental.pallas import tpu_sc as plsc

LAT_BINS = 1800
LON_BINS = 3600
N_EMBD = 384
B = 4096
L = 50

NC, NS, LANES = 2, 16, 16
NW = NC * NS
N = B * L
PER_W = N // NW
CHUNK = 64
NCHUNK = PER_W // CHUNK

_mesh = plsc.VectorSubcoreMesh(core_axis_name="c", subcore_axis_name="s")


@functools.partial(
    pl.kernel,
    out_type=jax.ShapeDtypeStruct((N, N_EMBD), jnp.float32),
    mesh=_mesh,
    scratch_types=[
        pltpu.VMEM((PER_W,), jnp.float32),
        pltpu.VMEM((PER_W,), jnp.int32),
        pltpu.VMEM((PER_W,), jnp.int32),
        pltpu.VMEM((CHUNK, N_EMBD), jnp.float32),
        pltpu.VMEM((CHUNK, N_EMBD), jnp.float32),
        pltpu.VMEM((CHUNK, N_EMBD), jnp.float32),
        pltpu.VMEM((CHUNK, N_EMBD), jnp.float32),
        pltpu.SemaphoreType.DMA,
        pltpu.SemaphoreType.DMA,
        pltpu.SemaphoreType.DMA,
        pltpu.SemaphoreType.DMA,
        pltpu.SemaphoreType.DMA,
        pltpu.SemaphoreType.DMA,
    ],
)
def _sc_embed(lat_hbm, lon_hbm, lat_tab, lon_tab, out_hbm,
              coords_v, ilat_v, ilon_v,
              ga0, ga1, gb0, gb1,
              sga0, sga1, sgb0, sgb1, swb0, swb1):
    wid = lax.axis_index("s") * NC + lax.axis_index("c")
    base = wid * PER_W

    ga = (ga0, ga1)
    gb = (gb0, gb1)
    sga = (sga0, sga1)
    sgb = (sgb0, sgb1)
    swb = (swb0, swb1)

    pltpu.sync_copy(lat_hbm.at[pl.ds(base, PER_W)], coords_v)

    def lat_idx_body(g, carry):
        sl = pl.ds(g * LANES, LANES)
        i = ((coords_v[sl] + 90.0) * (LAT_BINS / 180.0)).astype(jnp.int32)
        ilat_v[sl] = jnp.minimum(jnp.maximum(i, 0), LAT_BINS - 1)
        return carry

    lax.fori_loop(0, PER_W // LANES, lat_idx_body, 0, unroll=False)

    pltpu.sync_copy(lon_hbm.at[pl.ds(base, PER_W)], coords_v)

    def lon_idx_body(g, carry):
        sl = pl.ds(g * LANES, LANES)
        i = ((coords_v[sl] + 180.0) * (LON_BINS / 360.0)).astype(jnp.int32)
        ilon_v[sl] = jnp.minimum(jnp.maximum(i, 0), LON_BINS - 1)
        return carry

    lax.fori_loop(0, PER_W // LANES, lon_idx_body, 0, unroll=False)

    def issue_gathers(ci, b):
        isl = pl.ds(ci * CHUNK, CHUNK)
        pltpu.async_copy(lat_tab.at[ilat_v.at[isl]], ga[b], sga[b])
        pltpu.async_copy(lon_tab.at[ilon_v.at[isl]], gb[b], sgb[b])

    def wait_gathers(b):
        pltpu.make_async_copy(lat_tab.at[pl.ds(0, CHUNK)], ga[b], sga[b]).wait()
        pltpu.make_async_copy(lon_tab.at[pl.ds(0, CHUNK)], gb[b], sgb[b]).wait()

    def issue_wb(ci, b):
        pltpu.async_copy(ga[b], out_hbm.at[pl.ds(base + ci * CHUNK, CHUNK)],
                         swb[b])

    def wait_wb(b):
        pltpu.make_async_copy(ga[b], out_hbm.at[pl.ds(0, CHUNK)], swb[b]).wait()

    issue_gathers(0, 0)

    def pair_body(pi, carry):
        for b in (0, 1):
            ci = pi * 2 + b
            nxt = ci + 1
            other = 1 - b

            @pl.when(nxt < NCHUNK)
            def _issue_next():
                @pl.when(nxt >= 2)
                def _drain_wb():
                    wait_wb(other)

                issue_gathers(nxt, other)

            wait_gathers(b)

            def add_row(r, carry2):
                for j in range(N_EMBD // LANES):
                    sl = pl.ds(j * LANES, LANES)
                    plsc.addupdate(ga[b].at[r, sl], gb[b][r, sl])
                return carry2

            lax.fori_loop(0, CHUNK, add_row, 0, unroll=2)

            issue_wb(ci, b)
        return carry

    lax.fori_loop(0, NCHUNK // 2, pair_body, 0, unroll=False)

    wait_wb(0)
    wait_wb(1)


def kernel(lat, lon, lat_table, lon_table):
    lat_flat = lat.reshape(N)
    lon_flat = lon.reshape(N)
    out = _sc_embed(lat_flat, lon_flat, lat_table, lon_table)
    return out.reshape(B, L, N_EMBD)


# speedup vs baseline: 4.6362x; 1.3706x over previous
import functools

import jax
import jax.numpy as jnp
from jax import lax
from jax.experimental import pallas as pl
from jax.experimental.pallas import tpu as pltpu
from jax.experimental.pallas import tpu_sc as plsc

LAT_BINS = 1800
LON_BINS = 3600
N_EMBD = 384
B = 4096
L = 50

NC, NS, LANES = 2, 16, 16
NW = NC * NS
N = B * L
PER_W = N // NW
B_PER_W = B // NW
LTILE = 56
LPAD = 64
GPB = LPAD // LANES

_mesh = plsc.VectorSubcoreMesh(core_axis_name="c", subcore_axis_name="s")


@functools.partial(
    pl.kernel,
    out_type=jax.ShapeDtypeStruct((B, LTILE, N_EMBD), jnp.float32),
    mesh=_mesh,
    scratch_types=[
        pltpu.VMEM((PER_W + LANES,), jnp.float32),
        pltpu.VMEM((B_PER_W * LPAD,), jnp.int32),
        pltpu.VMEM((B_PER_W * LPAD,), jnp.int32),
        pltpu.VMEM((1, LTILE, N_EMBD), jnp.float32),
        pltpu.VMEM((1, LTILE, N_EMBD), jnp.float32),
        pltpu.VMEM((1, LTILE, N_EMBD), jnp.float32),
        pltpu.VMEM((1, LTILE, N_EMBD), jnp.float32),
        pltpu.SemaphoreType.DMA,
        pltpu.SemaphoreType.DMA,
        pltpu.SemaphoreType.DMA,
        pltpu.SemaphoreType.DMA,
        pltpu.SemaphoreType.DMA,
        pltpu.SemaphoreType.DMA,
    ],
)
def _sc_embed(lat_hbm, lon_hbm, lat_tab, lon_tab, out_hbm,
              coords_v, ilat_v, ilon_v,
              ga0, ga1, gb0, gb1,
              sga0, sga1, sgb0, sgb1, swb0, swb1):
    wid = lax.axis_index("s") * NC + lax.axis_index("c")
    base = wid * PER_W
    bbase = wid * B_PER_W

    ga = (ga0, ga1)
    gb = (gb0, gb1)
    sga = (sga0, sga1)
    sgb = (sgb0, sgb1)
    swb = (swb0, swb1)

    pltpu.sync_copy(lat_hbm.at[pl.ds(base, PER_W)],
                    coords_v.at[pl.ds(0, PER_W)])

    def lat_idx_body(bl, carry):
        for g in range(GPB):
            c = coords_v[pl.ds(bl * L + g * LANES, LANES)]
            i = ((c + 90.0) * (LAT_BINS / 180.0)).astype(jnp.int32)
            ilat_v[pl.ds(bl * LPAD + g * LANES, LANES)] = jnp.minimum(
                jnp.maximum(i, 0), LAT_BINS - 1)
        return carry

    lax.fori_loop(0, B_PER_W, lat_idx_body, 0, unroll=False)

    pltpu.sync_copy(lon_hbm.at[pl.ds(base, PER_W)],
                    coords_v.at[pl.ds(0, PER_W)])

    def lon_idx_body(bl, carry):
        for g in range(GPB):
            c = coords_v[pl.ds(bl * L + g * LANES, LANES)]
            i = ((c + 180.0) * (LON_BINS / 360.0)).astype(jnp.int32)
            ilon_v[pl.ds(bl * LPAD + g * LANES, LANES)] = jnp.minimum(
                jnp.maximum(i, 0), LON_BINS - 1)
        return carry

    lax.fori_loop(0, B_PER_W, lon_idx_body, 0, unroll=False)

    def issue_gathers(ci, b):
        isl = pl.ds(ci * LPAD, LTILE)
        pltpu.async_copy(lat_tab.at[ilat_v.at[isl]], ga[b].at[0], sga[b])
        pltpu.async_copy(lon_tab.at[ilon_v.at[isl]], gb[b].at[0], sgb[b])

    def wait_gathers(b):
        pltpu.make_async_copy(out_hbm.at[0], ga[b].at[0], sga[b]).wait()
        pltpu.make_async_copy(out_hbm.at[0], gb[b].at[0], sgb[b]).wait()

    def issue_wb(ci, b):
        pltpu.async_copy(ga[b], out_hbm.at[pl.ds(bbase + ci, 1)], swb[b])

    def wait_wb(b):
        pltpu.make_async_copy(ga[b], out_hbm.at[pl.ds(0, 1)], swb[b]).wait()

    issue_gathers(0, 0)

    def pair_body(pi, carry):
        for b in (0, 1):
            ci = pi * 2 + b
            nxt = ci + 1
            other = 1 - b

            @pl.when(nxt < B_PER_W)
            def _issue_next():
                @pl.when(nxt >= 2)
                def _drain_wb():
                    wait_wb(other)

                issue_gathers(nxt, other)

            wait_gathers(b)

            def add_row(r, carry2):
                for j in range(N_EMBD // LANES):
                    sl = pl.ds(j * LANES, LANES)
                    plsc.addupdate(ga[b].at[0, r, sl], gb[b][0, r, sl])
                return carry2

            lax.fori_loop(0, L, add_row, 0, unroll=2)

            issue_wb(ci, b)
        return carry

    lax.fori_loop(0, B_PER_W // 2, pair_body, 0, unroll=False)

    wait_wb(0)
    wait_wb(1)


def kernel(lat, lon, lat_table, lon_table):
    lat_flat = lat.reshape(N)
    lon_flat = lon.reshape(N)
    out = _sc_embed(lat_flat, lon_flat, lat_table, lon_table)
    return out[:, :L, :]
